# Initial kernel scaffold; baseline (speedup 1.0000x reference)
import jax
import jax.numpy as jnp
from jax.experimental import pallas as pl


def kernel(snp, snp_ids, segment_ids, filters, W1, b1, bn1_w, bn1_b, W2, b2, bn2_w, bn2_b, Wm, bm):
    raise NotImplementedError("write your pallas kernel here")


# speedup vs baseline: 19.9786x; 19.9786x over previous
import functools
import math

import jax
import jax.numpy as jnp
from jax import lax
from jax.experimental import pallas as pl
from jax.experimental.pallas import tpu as pltpu
from jax.experimental.pallas import tpu_sc as plsc

B = 16
N_SNPS = 100000
N_NODES = 300000
N_GENES = 18000
NF = 8
DH = 64
FD = 16
MAIN_DIM = 15

NC = 2
NS = 16
NW = NC * NS
CK = 128
NCHUNK = -(-N_NODES // (NW * CK))
NPAD = NW * NCHUNK * CK
ACC_ROWS = 18432
ZROWS = ACC_ROWS // NS

_BN_SCALE = 1.0 / math.sqrt(1.0 + 1e-5)


def _scale_body(snp_ref, filt_ref, out_ref):
    fbar = jnp.mean(filt_ref[...], axis=0, keepdims=True)
    out_ref[...] = jnp.transpose(snp_ref[...] * fbar)


_scale = pl.pallas_call(
    _scale_body,
    out_shape=jax.ShapeDtypeStruct((N_SNPS, B), jnp.float32),
)


@functools.partial(
    pl.kernel,
    out_type=jax.ShapeDtypeStruct((NC, ACC_ROWS, B), jnp.float32),
    mesh=plsc.VectorSubcoreMesh(core_axis_name="c", subcore_axis_name="s"),
    scratch_types=[
        pltpu.VMEM((NCHUNK, CK), jnp.int32),
        pltpu.VMEM((NCHUNK, CK), jnp.int32),
        pltpu.VMEM((CK, B), jnp.float32),
        pltpu.VMEM((CK, B), jnp.float32),
        pltpu.VMEM_SHARED((ACC_ROWS, B), jnp.float32),
        pltpu.SemaphoreType.DMA,
    ],
    compiler_params=pltpu.CompilerParams(use_tc_tiling_on_sc=False),
)
def _sc_segsum(tab_hbm, ids_hbm, segs_hbm, out_hbm,
               idx_v, seg_v, rows_v, zbuf, acc, sem):
    c = lax.axis_index("c")
    s = lax.axis_index("s")
    wid = s * NC + c

    def _zrow(i, _):
        zbuf[i, :] = jnp.zeros((B,), jnp.float32)
        return 0
    lax.fori_loop(0, CK, _zrow, 0)

    def _zacc(k, _):
        pltpu.sync_copy(zbuf, acc.at[pl.ds(s * ZROWS + k * CK, CK)])
        return 0
    lax.fori_loop(0, ZROWS // CK, _zacc, 0)

    pltpu.sync_copy(ids_hbm.at[wid], idx_v)
    pltpu.sync_copy(segs_hbm.at[wid], seg_v)

    plsc.subcore_barrier()

    def _step(j, _):
        pltpu.async_copy(tab_hbm.at[idx_v.at[j]], rows_v, sem).wait()
        pltpu.sync_copy(rows_v, acc.at[seg_v.at[j]], add=True)
        return 0
    lax.fori_loop(0, NCHUNK, _step, 0)

    plsc.subcore_barrier()
    pltpu.sync_copy(acc.at[pl.ds(s * ZROWS, ZROWS)],
                    out_hbm.at[c, pl.ds(s * ZROWS, ZROWS)])


def _mlp_body(parts_ref, W1_ref, b1_ref, g1_ref, be1_ref,
              W2_ref, b2_ref, g2_ref, be2_ref, Wm_ref, bm_ref, out_ref):
    p = parts_ref[0] + parts_ref[1]
    h = lax.dot_general(W1_ref[...], p, (((1,), (0,)), ((), ())),
                        preferred_element_type=jnp.float32)
    h = h + b1_ref[...]
    h = h * (g1_ref[...] * _BN_SCALE) + be1_ref[...]
    h = jnp.maximum(h, 0.0)
    h = lax.dot_general(W2_ref[...], h, (((1,), (0,)), ((), ())),
                        preferred_element_type=jnp.float32)
    h = h + b2_ref[...]
    h = h * (g2_ref[...] * _BN_SCALE) + be2_ref[...]
    h = jnp.maximum(h, 0.0)
    out_ref[...] = lax.dot_general(h, Wm_ref[...], (((0,), (1,)), ((), ())),
                                   preferred_element_type=jnp.float32) + bm_ref[...]


_mlp = pl.pallas_call(
    _mlp_body,
    out_shape=jax.ShapeDtypeStruct((B, 1), jnp.float32),
)


def kernel(snp, snp_ids, segment_ids, filters,
           W1, b1, bn1_w, bn1_b, W2, b2, bn2_w, bn2_b, Wm, bm):
    tab = _scale(snp, filters)
    pad = NPAD - N_NODES
    ids3 = jnp.concatenate(
        [snp_ids, jnp.zeros((pad,), jnp.int32)]).reshape(NW, NCHUNK, CK)
    segs3 = jnp.concatenate(
        [segment_ids, jnp.full((pad,), N_GENES, jnp.int32)]).reshape(NW, NCHUNK, CK)
    parts = _sc_segsum(tab, ids3, segs3)
    Wm_p = jnp.pad(Wm, ((0, 0), (0, FD - MAIN_DIM)))
    W1_p = jnp.pad(W1, ((0, 0), (0, ACC_ROWS - N_GENES)))
    return _mlp(parts, W1_p,
                b1.reshape(DH, 1), bn1_w.reshape(DH, 1), bn1_b.reshape(DH, 1),
                W2, b2.reshape(FD, 1), bn2_w.reshape(FD, 1), bn2_b.reshape(FD, 1),
                Wm_p, bm.reshape(1, 1))


# speedup vs baseline: 20.1415x; 1.0082x over previous
import functools
import math

import jax
import jax.numpy as jnp
from jax import lax
from jax.experimental import pallas as pl
from jax.experimental.pallas import tpu as pltpu
from jax.experimental.pallas import tpu_sc as plsc

B = 16
N_SNPS = 100000
N_NODES = 300000
N_GENES = 18000
NF = 8
DH = 64
FD = 16
MAIN_DIM = 15

NC = 2
NS = 16
NW = NC * NS
CK = 128
NPH = 4
PH = 19
NCHUNK = NPH * PH
NPAD = NW * NCHUNK * CK
ACC_ROWS = 18432
ZROWS = ACC_ROWS // NS

_BN_SCALE = 1.0 / math.sqrt(1.0 + 1e-5)


def _scale_body(snp_ref, filt_ref, out_ref):
    fbar = jnp.mean(filt_ref[...], axis=0, keepdims=True)
    out_ref[...] = jnp.transpose(snp_ref[...] * fbar)


_scale = pl.pallas_call(
    _scale_body,
    out_shape=jax.ShapeDtypeStruct((N_SNPS, B), jnp.float32),
)


@functools.partial(
    pl.kernel,
    out_type=jax.ShapeDtypeStruct((NC, ACC_ROWS, B), jnp.float32),
    mesh=plsc.VectorSubcoreMesh(core_axis_name="c", subcore_axis_name="s"),
    scratch_types=[
        pltpu.VMEM((NCHUNK, CK), jnp.int32),
        pltpu.VMEM((NCHUNK, CK), jnp.int32),
        pltpu.VMEM((2, PH, CK, B), jnp.float32),
        pltpu.VMEM((CK, B), jnp.float32),
        pltpu.VMEM_SHARED((ACC_ROWS, B), jnp.float32),
        pltpu.SemaphoreType.DMA,
        pltpu.SemaphoreType.DMA,
    ],
    compiler_params=pltpu.CompilerParams(use_tc_tiling_on_sc=False),
)
def _sc_segsum(tab_hbm, ids_hbm, segs_hbm, out_hbm,
               idx_v, seg_v, rows_v, zbuf, acc, gsem, ssem):
    c = lax.axis_index("c")
    s = lax.axis_index("s")
    wid = s * NC + c

    def _zrow(i, _):
        zbuf[i, :] = jnp.zeros((B,), jnp.float32)
        return 0
    lax.fori_loop(0, CK, _zrow, 0)

    def _zacc(k, _):
        pltpu.sync_copy(zbuf, acc.at[pl.ds(s * ZROWS + k * CK, CK)])
        return 0
    lax.fori_loop(0, ZROWS // CK, _zacc, 0)

    pltpu.sync_copy(ids_hbm.at[wid], idx_v)
    pltpu.sync_copy(segs_hbm.at[wid], seg_v)

    plsc.subcore_barrier()

    def _fire_g(ph, st):
        def f(r, _):
            pltpu.async_copy(tab_hbm.at[idx_v.at[ph * PH + r]],
                             rows_v.at[st, r], gsem)
            return 0
        lax.fori_loop(0, PH, f, 0)

    def _drain_g(ph, st):
        def f(r, _):
            pltpu.make_async_copy(tab_hbm.at[idx_v.at[ph * PH + r]],
                                  rows_v.at[st, r], gsem).wait()
            return 0
        lax.fori_loop(0, PH, f, 0)

    def _fire_s(ph, st):
        def f(r, _):
            pltpu.async_copy(rows_v.at[st, r],
                             acc.at[seg_v.at[ph * PH + r]], ssem, add=True)
            return 0
        lax.fori_loop(0, PH, f, 0)

    def _drain_s(ph, st):
        def f(r, _):
            pltpu.make_async_copy(rows_v.at[st, r],
                                  acc.at[seg_v.at[ph * PH + r]], ssem).wait()
            return 0
        lax.fori_loop(0, PH, f, 0)

    _fire_g(0, 0)
    for ph in range(NPH):
        st = ph % 2
        _drain_g(ph, st)
        if ph > 0:
            _drain_s(ph - 1, 1 - st)
        _fire_s(ph, st)
        if ph + 1 < NPH:
            _fire_g(ph + 1, 1 - st)
    _drain_s(NPH - 1, (NPH - 1) % 2)

    plsc.subcore_barrier()
    pltpu.sync_copy(acc.at[pl.ds(s * ZROWS, ZROWS)],
                    out_hbm.at[c, pl.ds(s * ZROWS, ZROWS)])


def _mlp_body(parts_ref, W1_ref, b1_ref, g1_ref, be1_ref,
              W2_ref, b2_ref, g2_ref, be2_ref, Wm_ref, bm_ref, out_ref):
    p = parts_ref[0] + parts_ref[1]
    h = lax.dot_general(W1_ref[...], p, (((1,), (0,)), ((), ())),
                        preferred_element_type=jnp.float32)
    h = h + b1_ref[...]
    h = h * (g1_ref[...] * _BN_SCALE) + be1_ref[...]
    h = jnp.maximum(h, 0.0)
    h = lax.dot_general(W2_ref[...], h, (((1,), (0,)), ((), ())),
                        preferred_element_type=jnp.float32)
    h = h + b2_ref[...]
    h = h * (g2_ref[...] * _BN_SCALE) + be2_ref[...]
    h = jnp.maximum(h, 0.0)
    out_ref[...] = lax.dot_general(h, Wm_ref[...], (((0,), (1,)), ((), ())),
                                   preferred_element_type=jnp.float32) + bm_ref[...]


_mlp = pl.pallas_call(
    _mlp_body,
    out_shape=jax.ShapeDtypeStruct((B, 1), jnp.float32),
)


def kernel(snp, snp_ids, segment_ids, filters,
           W1, b1, bn1_w, bn1_b, W2, b2, bn2_w, bn2_b, Wm, bm):
    tab = _scale(snp, filters)
    pad = NPAD - N_NODES
    ids3 = jnp.concatenate(
        [snp_ids, jnp.zeros((pad,), jnp.int32)]).reshape(NW, NCHUNK, CK)
    segs3 = jnp.concatenate(
        [segment_ids, jnp.full((pad,), N_GENES, jnp.int32)]).reshape(NW, NCHUNK, CK)
    parts = _sc_segsum(tab, ids3, segs3)
    Wm_p = jnp.pad(Wm, ((0, 0), (0, FD - MAIN_DIM)))
    W1_p = jnp.pad(W1, ((0, 0), (0, ACC_ROWS - N_GENES)))
    return _mlp(parts, W1_p,
                b1.reshape(DH, 1), bn1_w.reshape(DH, 1), bn1_b.reshape(DH, 1),
                W2, b2.reshape(FD, 1), bn2_w.reshape(FD, 1), bn2_b.reshape(FD, 1),
                Wm_p, bm.reshape(1, 1))
